# Initial kernel scaffold; baseline (speedup 1.0000x reference)
import jax
import jax.numpy as jnp
from jax.experimental import pallas as pl


def kernel(x, router_w, head_centroids, temperature, input_experts, output_experts):
    raise NotImplementedError("write your pallas kernel here")


# speedup vs baseline: 3.0983x; 3.0983x over previous
import math

import jax
import jax.numpy as jnp
from jax.experimental import pallas as pl
from jax.experimental.pallas import tpu as pltpu

B, S, D = 2, 2048, 1024
N, HID, HD, G = 16, 64, 64, 64
T = B * S
TM = 512


def _erf(x):
    a1, a2, a3, a4, a5 = 0.254829592, -0.284496736, 1.421413741, -1.453152027, 1.061405429
    p = 0.3275911
    s = jnp.sign(x)
    ax = jnp.abs(x)
    t = 1.0 / (1.0 + p * ax)
    y = 1.0 - (((((a5 * t + a4) * t + a3) * t + a2) * t + a1) * t) * jnp.exp(-ax * ax)
    return s * y


def _gelu(x):
    return 0.5 * x * (1.0 + _erf(x * (1.0 / math.sqrt(2.0))))


def _body(temp_ref, x_ref, rwt_ref, ct_ref, win2_ref, wout_ref, emat_ref, out_ref):
    xt = x_ref[:]
    xb = xt.astype(jnp.bfloat16)

    z = jnp.dot(xb, rwt_ref[:].astype(jnp.bfloat16),
                preferred_element_type=jnp.float32)
    zn = z / jnp.maximum(jnp.sqrt(jnp.sum(z * z, axis=1, keepdims=True)), 1e-12)
    ct = ct_ref[:]
    cn = ct / jnp.maximum(jnp.sqrt(jnp.sum(ct * ct, axis=0, keepdims=True)), 1e-12)
    logits = jnp.dot(zn.astype(jnp.bfloat16), cn.astype(jnp.bfloat16),
                     preferred_element_type=jnp.float32) / math.sqrt(G)
    logits = logits * jnp.exp(temp_ref[0, 0])

    nidx = jax.lax.broadcasted_iota(jnp.int32, (TM, N), 1)
    neg = jnp.float32(-1e30)
    m1 = jnp.max(logits, axis=1, keepdims=True)
    i1 = jnp.min(jnp.where(logits == m1, nidx, N), axis=1, keepdims=True)
    l2 = jnp.where(nidx == i1, neg, logits)
    m2 = jnp.max(l2, axis=1, keepdims=True)
    i2 = jnp.min(jnp.where(l2 == m2, nidx, N), axis=1, keepdims=True)

    e = jnp.exp(logits - m1)
    probs = e / jnp.sum(e, axis=1, keepdims=True)
    p1 = jnp.max(probs, axis=1, keepdims=True)
    p2 = jnp.max(jnp.where(nidx == i1, neg, probs), axis=1, keepdims=True)
    w = jnp.where(nidx == i1, p1, 0.0) + jnp.where(nidx == i2, p2, 0.0)
    wexp = jnp.dot(w, emat_ref[:], preferred_element_type=jnp.float32)

    hidden = jnp.dot(xb, win2_ref[:].astype(jnp.bfloat16),
                     preferred_element_type=jnp.float32)
    hb = _gelu(hidden).astype(jnp.bfloat16)
    for n in range(N):
        out_ref[:, n * HD:(n + 1) * HD] = jnp.dot(
            hb[:, n * HID:(n + 1) * HID], wout_ref[n].astype(jnp.bfloat16),
            preferred_element_type=jnp.float32) * wexp[:, n * HD:(n + 1) * HD]


def kernel(x, router_w, head_centroids, temperature, input_experts, output_experts):
    xf = x.reshape(T, D)
    rwt = router_w.T
    ct = head_centroids.T
    win2 = input_experts.transpose(1, 0, 2).reshape(D, N * HID)
    head_of = jax.lax.broadcasted_iota(jnp.int32, (N, N * HD), 1) // HD
    emat = (head_of == jax.lax.broadcasted_iota(jnp.int32, (N, N * HD), 0)).astype(jnp.float32)
    temp = temperature.reshape(1, 1)

    out = pl.pallas_call(
        _body,
        grid=(T // TM,),
        in_specs=[
            pl.BlockSpec(memory_space=pltpu.SMEM),
            pl.BlockSpec((TM, D), lambda i: (i, 0)),
            pl.BlockSpec((D, G), lambda i: (0, 0)),
            pl.BlockSpec((G, N), lambda i: (0, 0)),
            pl.BlockSpec((D, N * HID), lambda i: (0, 0)),
            pl.BlockSpec((N, HID, HD), lambda i: (0, 0, 0)),
            pl.BlockSpec((N, N * HD), lambda i: (0, 0)),
        ],
        out_specs=pl.BlockSpec((TM, N * HD), lambda i: (i, 0)),
        out_shape=jax.ShapeDtypeStruct((T, N * HD), jnp.float32),
        compiler_params=pltpu.CompilerParams(dimension_semantics=("arbitrary",)),
    )(temp, xf, rwt, ct, win2, output_experts, emat)
    return out.reshape(B, S, N * HD)
